# baseline (speedup 1.0000x reference)
import functools
import jax
import jax.numpy as jnp
from jax import lax
from jax.experimental import pallas as pl
from jax.experimental.pallas import tpu as pltpu, tpu_sc as plsc

EMBED = 64
BATCH = 16384
HIST = 50
VOCAB = 1000000
NC = 2
NS = 16
NW = NC * NS

BB = BATCH // 128
UNITS = HIST * BB
UPW = UNITS // NW
NPAIR = UPW // 2


VB = 7813
RT_ROWS = 500032
APW = 124


@jax.jit
def _sc_table_fmt(tableT):
    mesh = plsc.VectorSubcoreMesh(core_axis_name="c", subcore_axis_name="s")

    @functools.partial(
        pl.kernel,
        out_type=jax.ShapeDtypeStruct((RT_ROWS, 128), jnp.float32),
        mesh=mesh,
        scratch_types=[
            pltpu.VMEM((2, EMBED, 256), jnp.float32),
            pltpu.VMEM((2, 128, 128), jnp.float32),
            pltpu.SemaphoreType.DMA,
            pltpu.SemaphoreType.DMA,
            pltpu.SemaphoreType.DMA,
            pltpu.SemaphoreType.DMA,
        ],
        compiler_params=pltpu.CompilerParams(
            needs_layout_passes=False, disable_bounds_checks=True),
    )
    def k(tt_hbm, rt_hbm, tin_v, tout_v, sem_i0, sem_i1, sem_o0, sem_o1):
        cid = lax.axis_index("c")
        sid = lax.axis_index("s")
        wid = sid * NC + cid
        iota = lax.broadcasted_iota(jnp.int32, (16,), 0)
        vcols = [vc * 16 + iota for vc in range(16)]
        sr0 = iota >> 1
        pc = (iota & 1) * 64
        sem_i = (sem_i0, sem_i1)
        sem_o = (sem_o0, sem_o1)

        def vb_of(u):
            return jnp.minimum((wid * APW + u) * 2, VB - 2)

        def fire_in(u, b):
            vb = vb_of(u)
            for eb in range(8):
                pltpu.async_copy(
                    tt_hbm.at[pl.ds(eb * 8, 8), pl.ds(vb * 128, 256)],
                    tin_v.at[b, pl.ds(eb * 8, 8), :],
                    sem_i[b],
                )

        def drain_in(b):
            pltpu.make_async_copy(rt_hbm.at[pl.ds(0, 128)], tin_v.at[b],
                                  sem_i[b]).wait()

        def drain_out(b):
            pltpu.make_async_copy(rt_hbm.at[pl.ds(0, 128)], tout_v.at[b],
                                  sem_o[b]).wait()

        def transpose_unit(b):
            tin_b = tin_v.at[b]
            tout_b = tout_v.at[b]

            def estep(e0, _):
                emod = (e0 + iota) & 15
                for ec in range(EMBED // 16):
                    erows = ec * 16 + emod
                    scols = pc + erows
                    for j in range(16):
                        vals = plsc.load_gather(tin_b, [erows, vcols[j]])
                        plsc.store_scatter(tout_b, [sr0 + j * 8, scols],
                                           vals)
                return 0

            lax.fori_loop(0, 16, estep, 0)

        def fire_out(u, b):
            vb = vb_of(u)
            pltpu.async_copy(tout_v.at[b],
                             rt_hbm.at[pl.ds(vb * 64, 128), :], sem_o[b])

        fire_in(0, 0)

        def pair(p, _):
            for b in range(2):
                u = 2 * p + b
                drain_in(b)
                if b == 0:
                    fire_in(u + 1, 1)
                else:
                    @pl.when(p + 1 < APW // 2)
                    def _():
                        fire_in(u + 1, 0)

                @pl.when(p > 0)
                def _():
                    drain_out(b)
                transpose_unit(b)
                fire_out(u, b)
            return 0

        lax.fori_loop(0, APW // 2, pair, 0)
        drain_out(0)
        drain_out(1)

    return k(tableT)


@jax.jit
def _sc_gather_fmt(rt, idsr):
    mesh = plsc.VectorSubcoreMesh(core_axis_name="c", subcore_axis_name="s")

    @functools.partial(
        pl.kernel,
        out_type=jax.ShapeDtypeStruct((HIST, EMBED, BATCH), jnp.float32),
        mesh=mesh,
        scratch_types=[
            pltpu.VMEM((UPW, 128), jnp.int32),
            pltpu.VMEM((2, 128), jnp.int32),
            pltpu.VMEM((2, 128, 128), jnp.float32),
            pltpu.VMEM((2, EMBED, 128), jnp.float32),
            pltpu.SemaphoreType.DMA,
            pltpu.SemaphoreType.DMA,
            pltpu.SemaphoreType.DMA,
            pltpu.SemaphoreType.DMA,
        ],
        compiler_params=pltpu.CompilerParams(needs_layout_passes=False),
    )
    def k(ids_hbm, rt_hbm, out_hbm, idx_v, pidx_v, staged_v, outs_v,
          sem_g0, sem_g1, sem_o0, sem_o1):
        cid = lax.axis_index("c")
        sid = lax.axis_index("s")
        wid = sid * NC + cid
        pltpu.sync_copy(ids_hbm.at[pl.ds(wid * UPW, UPW)], idx_v)
        iota = lax.broadcasted_iota(jnp.int32, (16,), 0)
        rows = [bc * 16 + iota for bc in range(8)]
        sem_g = (sem_g0, sem_g1)
        sem_o = (sem_o0, sem_o1)

        def fire_gather(u, b):
            for bc in range(8):
                v16 = idx_v[u, pl.ds(bc * 16, 16)]
                pidx_v[b, pl.ds(bc * 16, 16)] = v16 >> 1
            pltpu.async_copy(rt_hbm.at[pidx_v.at[b]], staged_v.at[b], sem_g[b])

        def drain_gather(b):
            pltpu.make_async_copy(rt_hbm.at[pl.ds(0, 128)], staged_v.at[b],
                                  sem_g[b]).wait()

        def drain_outs(b):
            pltpu.make_async_copy(rt_hbm.at[pl.ds(0, 32)], outs_v.at[b],
                                  sem_o[b]).wait()

        def transpose_unit(u, b):
            halves = [(idx_v[u, pl.ds(bc * 16, 16)] & 1) * 64
                      for bc in range(8)]
            staged_b = staged_v.at[b]
            outs_b = outs_v.at[b]

            def estep(e0, _):
                emod = (e0 + iota) & 15
                for ec in range(EMBED // 16):
                    erows = ec * 16 + emod
                    for bc in range(8):
                        vals = plsc.load_gather(
                            staged_b, [rows[bc], halves[bc] + erows])
                        plsc.store_scatter(outs_b, [erows, rows[bc]], vals)
                return 0

            lax.fori_loop(0, 16, estep, 0)

        def fire_outs(u, b):
            unit_id = wid * UPW + u
            h = unit_id // BB
            bb = unit_id % BB
            for eb in range(8):
                pltpu.async_copy(
                    outs_v.at[b, pl.ds(eb * 8, 8), :],
                    out_hbm.at[h, pl.ds(eb * 8, 8), pl.ds(bb * 128, 128)],
                    sem_o[b],
                )

        fire_gather(0, 0)

        def pair(p, _):
            for b in range(2):
                u = 2 * p + b
                drain_gather(b)
                if b == 0:
                    fire_gather(u + 1, 1)
                else:
                    @pl.when(p + 1 < NPAIR)
                    def _():
                        fire_gather(u + 1, 0)

                @pl.when(p > 0)
                def _():
                    drain_outs(b)
                transpose_unit(u, b)
                fire_outs(u, b)
            return 0

        lax.fori_loop(0, NPAIR, pair, 0)
        drain_outs(0)
        drain_outs(1)

    return k(idsr, rt)


def kernel(ids, table):
    rt = _sc_table_fmt(table.T)
    idsr = ids.astype(jnp.int32).T.reshape(UNITS, 128)
    ot = _sc_gather_fmt(rt, idsr)
    return ot.transpose(2, 0, 1)


# speedup vs baseline: 1.1683x; 1.1683x over previous
import functools
import jax
import jax.numpy as jnp
from jax import lax
from jax.experimental import pallas as pl
from jax.experimental.pallas import tpu as pltpu, tpu_sc as plsc

EMBED = 64
BATCH = 16384
HIST = 50
VOCAB = 1000000
NC = 2
NS = 16
NW = NC * NS

BB = BATCH // 128
UNITS = HIST * BB
UPW = UNITS // NW
NPAIR = UPW // 2


VB = 7813
RT_ROWS = 500032
APW = 246


@jax.jit
def _sc_table_fmt(tableT):
    mesh = plsc.VectorSubcoreMesh(core_axis_name="c", subcore_axis_name="s")

    @functools.partial(
        pl.kernel,
        out_type=jax.ShapeDtypeStruct((RT_ROWS, 128), jnp.float32),
        mesh=mesh,
        scratch_types=[
            pltpu.VMEM((2, EMBED, 128), jnp.float32),
            pltpu.VMEM((2, EMBED, 128), jnp.float32),
            pltpu.SemaphoreType.DMA,
            pltpu.SemaphoreType.DMA,
            pltpu.SemaphoreType.DMA,
            pltpu.SemaphoreType.DMA,
        ],
        compiler_params=pltpu.CompilerParams(
            needs_layout_passes=False, disable_bounds_checks=True),
    )
    def k(tt_hbm, rt_hbm, tin_v, tout_v, sem_i0, sem_i1, sem_o0, sem_o1):
        cid = lax.axis_index("c")
        sid = lax.axis_index("s")
        wid = sid * NC + cid
        iota = lax.broadcasted_iota(jnp.int32, (16,), 0)
        vcols = [vc * 16 + iota for vc in range(8)]
        srows = [(vc * 16 + iota) >> 1 for vc in range(8)]
        pcols = [((vc * 16 + iota) & 1) * 64 for vc in range(8)]
        sem_i = (sem_i0, sem_i1)
        sem_o = (sem_o0, sem_o1)

        def vb_of(u):
            return jnp.minimum(wid * APW + u, VB - 1)

        def fire_in(u, b):
            vb = vb_of(u)
            for eb in range(8):
                pltpu.async_copy(
                    tt_hbm.at[pl.ds(eb * 8, 8), pl.ds(vb * 128, 128)],
                    tin_v.at[b, pl.ds(eb * 8, 8), :],
                    sem_i[b],
                )

        def drain_in(b):
            pltpu.make_async_copy(rt_hbm.at[pl.ds(0, EMBED)], tin_v.at[b],
                                  sem_i[b]).wait()

        def drain_out(b):
            pltpu.make_async_copy(rt_hbm.at[pl.ds(0, EMBED)], tout_v.at[b],
                                  sem_o[b]).wait()

        def transpose_unit(b):
            tin_b = tin_v.at[b]
            tout_b = tout_v.at[b]

            def estep(e0, _):
                emod = (e0 + iota) & 15
                for ec in range(EMBED // 16):
                    erows = ec * 16 + emod
                    for vc in range(8):
                        vals = plsc.load_gather(tin_b, [erows, vcols[vc]])
                        plsc.store_scatter(tout_b,
                                           [srows[vc], pcols[vc] + erows],
                                           vals)
                return 0

            lax.fori_loop(0, 16, estep, 0)

        def fire_out(u, b):
            vb = vb_of(u)
            pltpu.async_copy(tout_v.at[b],
                             rt_hbm.at[pl.ds(vb * 64, EMBED), :], sem_o[b])

        fire_in(0, 0)

        def pair(p, _):
            for b in range(2):
                u = 2 * p + b
                drain_in(b)
                if b == 0:
                    fire_in(u + 1, 1)
                else:
                    @pl.when(p + 1 < APW // 2)
                    def _():
                        fire_in(u + 1, 0)

                @pl.when(p > 0)
                def _():
                    drain_out(b)
                transpose_unit(b)
                fire_out(u, b)
            return 0

        lax.fori_loop(0, APW // 2, pair, 0)
        drain_out(0)
        drain_out(1)

    return k(tableT)


@jax.jit
def _sc_gather_fmt(rt, idsr):
    mesh = plsc.VectorSubcoreMesh(core_axis_name="c", subcore_axis_name="s")

    @functools.partial(
        pl.kernel,
        out_type=jax.ShapeDtypeStruct((HIST, EMBED, BATCH), jnp.float32),
        mesh=mesh,
        scratch_types=[
            pltpu.VMEM((UPW, 128), jnp.int32),
            pltpu.VMEM((2, 128), jnp.int32),
            pltpu.VMEM((2, 128, 128), jnp.float32),
            pltpu.VMEM((2, EMBED, 128), jnp.float32),
            pltpu.SemaphoreType.DMA,
            pltpu.SemaphoreType.DMA,
            pltpu.SemaphoreType.DMA,
            pltpu.SemaphoreType.DMA,
        ],
        compiler_params=pltpu.CompilerParams(needs_layout_passes=False),
    )
    def k(ids_hbm, rt_hbm, out_hbm, idx_v, pidx_v, staged_v, outs_v,
          sem_g0, sem_g1, sem_o0, sem_o1):
        cid = lax.axis_index("c")
        sid = lax.axis_index("s")
        wid = sid * NC + cid
        pltpu.sync_copy(ids_hbm.at[pl.ds(wid * UPW, UPW)], idx_v)
        iota = lax.broadcasted_iota(jnp.int32, (16,), 0)
        rows = [bc * 16 + iota for bc in range(8)]
        sem_g = (sem_g0, sem_g1)
        sem_o = (sem_o0, sem_o1)

        def fire_gather(u, b):
            for bc in range(8):
                v16 = idx_v[u, pl.ds(bc * 16, 16)]
                pidx_v[b, pl.ds(bc * 16, 16)] = v16 >> 1
            pltpu.async_copy(rt_hbm.at[pidx_v.at[b]], staged_v.at[b], sem_g[b])

        def drain_gather(b):
            pltpu.make_async_copy(rt_hbm.at[pl.ds(0, 128)], staged_v.at[b],
                                  sem_g[b]).wait()

        def drain_outs(b):
            pltpu.make_async_copy(rt_hbm.at[pl.ds(0, 32)], outs_v.at[b],
                                  sem_o[b]).wait()

        def transpose_unit(u, b):
            halves = [(idx_v[u, pl.ds(bc * 16, 16)] & 1) * 64
                      for bc in range(8)]
            staged_b = staged_v.at[b]
            outs_b = outs_v.at[b]

            def estep(e0, _):
                emod = (e0 + iota) & 15
                for ec in range(EMBED // 16):
                    erows = ec * 16 + emod
                    for bc in range(8):
                        vals = plsc.load_gather(
                            staged_b, [rows[bc], halves[bc] + erows])
                        plsc.store_scatter(outs_b, [erows, rows[bc]], vals)
                return 0

            lax.fori_loop(0, 16, estep, 0)

        def fire_outs(u, b):
            unit_id = wid * UPW + u
            h = unit_id // BB
            bb = unit_id % BB
            for eb in range(8):
                pltpu.async_copy(
                    outs_v.at[b, pl.ds(eb * 8, 8), :],
                    out_hbm.at[h, pl.ds(eb * 8, 8), pl.ds(bb * 128, 128)],
                    sem_o[b],
                )

        fire_gather(0, 0)

        def pair(p, _):
            for b in range(2):
                u = 2 * p + b
                drain_gather(b)
                if b == 0:
                    fire_gather(u + 1, 1)
                else:
                    @pl.when(p + 1 < NPAIR)
                    def _():
                        fire_gather(u + 1, 0)

                @pl.when(p > 0)
                def _():
                    drain_outs(b)
                transpose_unit(u, b)
                fire_outs(u, b)
            return 0

        lax.fori_loop(0, NPAIR, pair, 0)
        drain_outs(0)
        drain_outs(1)

    return k(idsr, rt)


def kernel(ids, table):
    rt = _sc_table_fmt(table.T)
    idsr = ids.astype(jnp.int32).T.reshape(UNITS, 128)
    ot = _sc_gather_fmt(rt, idsr)
    return ot.transpose(2, 0, 1)
